# Initial kernel scaffold; baseline (speedup 1.0000x reference)
import jax
import jax.numpy as jnp
from jax.experimental import pallas as pl


def kernel(x, edge_index, edge_type, r, W):
    raise NotImplementedError("write your pallas kernel here")


# speedup vs baseline: 25.3840x; 25.3840x over previous
import functools

import jax
import jax.numpy as jnp
from jax import lax
from jax.experimental import pallas as pl
from jax.experimental.pallas import tpu as pltpu
from jax.experimental.pallas import tpu_sc as plsc

N = 10000
E = 320000
D = 128
NC = 2
NS = 16
NW = NC * NS
EPW = E // NW
B = 100
C = EPW // B
NP = 10240
RPT = NP // NS

_mesh = plsc.VectorSubcoreMesh(core_axis_name="c", subcore_axis_name="s")


@functools.partial(
    pl.kernel,
    mesh=_mesh,
    out_type=jax.ShapeDtypeStruct((NW, 1, N), jnp.float32),
    scratch_types=[
        pltpu.VMEM((1, EPW), jnp.int32),
        pltpu.VMEM((N,), jnp.float32),
    ],
    compiler_params=pltpu.CompilerParams(needs_layout_passes=False),
)
def _p1_degree(row_hbm, degp_hbm, idx_v, deg_v):
    c = lax.axis_index("c")
    s = lax.axis_index("s")
    wid = c * NS + s

    pltpu.sync_copy(row_hbm.at[wid], idx_v)

    def zero_body(i, _):
        deg_v[pl.ds(i * 16, 16)] = jnp.zeros((16,), jnp.float32)
        return 0

    lax.fori_loop(0, N // 16, zero_body, 0)

    ones16 = jnp.ones((16,), jnp.float32)

    def hist_body(j, _):
        idx = idx_v[0, pl.ds(j * 16, 16)]
        plsc.addupdate_scatter(deg_v, [idx], ones16)
        return 0

    lax.fori_loop(0, EPW // 16, hist_body, 0)
    pltpu.sync_copy(deg_v, degp_hbm.at[wid, 0])


@functools.partial(
    pl.kernel,
    mesh=_mesh,
    out_type=jax.ShapeDtypeStruct((NC, NP, D), jnp.float32),
    scratch_types=[
        pltpu.VMEM((C, B), jnp.int32),
        pltpu.VMEM((C, B), jnp.int32),
        pltpu.VMEM((B, D), jnp.float32),
        pltpu.VMEM((40, D), jnp.float32),
        pltpu.VMEM_SHARED((NP, D), jnp.float32),
        pltpu.SemaphoreType.DMA,
    ],
)
def _p3_scatter(y_hbm, col_hbm, row_hbm, zp_hbm, col_v, row_v, buf, zb_v,
                z_sh, sem):
    c = lax.axis_index("c")
    s = lax.axis_index("s")
    wid = c * NS + s

    pltpu.sync_copy(col_hbm.at[wid], col_v)
    pltpu.sync_copy(row_hbm.at[wid], row_v)
    for i in range(40):
        for k in range(D // 16):
            zb_v[i, pl.ds(k * 16, 16)] = jnp.zeros((16,), jnp.float32)

    def zero_body(i, _):
        pltpu.sync_copy(zb_v, z_sh.at[pl.ds(s * RPT + i * 40, 40)])
        return 0

    lax.fori_loop(0, RPT // 40, zero_body, 0)
    plsc.subcore_barrier()

    def edge_body(j, _):
        pltpu.async_copy(y_hbm.at[col_v.at[j]], buf, sem).wait()
        pltpu.sync_copy(buf, z_sh.at[row_v.at[j]], add=True)
        return 0

    lax.fori_loop(0, C, edge_body, 0)
    plsc.subcore_barrier()

    pltpu.sync_copy(z_sh.at[pl.ds(s * RPT, RPT)],
                    zp_hbm.at[c, pl.ds(s * RPT, RPT)])


_R = 400
_G = N // _R


def _deg_inv_block(degp_blk):
    d = jnp.sum(degp_blk, axis=0).reshape(_R, 1)
    safe = jnp.where(d > 0, d, 1.0)
    return jnp.where(d > 0, lax.rsqrt(safe), 0.0)


def _p2_body(x_ref, degp_ref, w_ref, y_ref):
    dinv = _deg_inv_block(degp_ref[:])
    y_ref[:] = jnp.dot(x_ref[:], w_ref[:],
                       preferred_element_type=jnp.float32) * dinv


def _p2_scale_matmul(x, degp4, W):
    return pl.pallas_call(
        _p2_body,
        grid=(_G,),
        in_specs=[
            pl.BlockSpec((_R, D), lambda i: (i, 0)),
            pl.BlockSpec((NW, 1, 1, _R), lambda i: (0, i, 0, 0)),
            pl.BlockSpec((D, D), lambda i: (0, 0)),
        ],
        out_specs=pl.BlockSpec((_R, D), lambda i: (i, 0)),
        out_shape=jax.ShapeDtypeStruct((N, D), jnp.float32),
    )(x, degp4, W)


def _p4_body(zp_ref, degp_ref, out_ref):
    dinv = _deg_inv_block(degp_ref[:])
    out_ref[:] = (zp_ref[0] + zp_ref[1]) * dinv


def _p4_combine(zp, degp4):
    return pl.pallas_call(
        _p4_body,
        grid=(_G,),
        in_specs=[
            pl.BlockSpec((NC, _R, D), lambda i: (0, i, 0)),
            pl.BlockSpec((NW, 1, 1, _R), lambda i: (0, i, 0, 0)),
        ],
        out_specs=pl.BlockSpec((_R, D), lambda i: (i, 0)),
        out_shape=jax.ShapeDtypeStruct((N, D), jnp.float32),
    )(zp, degp4)


def kernel(x, edge_index, edge_type, r, W):
    del edge_type, r
    ei = edge_index.astype(jnp.int32)
    row3 = ei[0].reshape(NW, C, B)
    col3 = ei[1].reshape(NW, C, B)
    degp = _p1_degree(ei[0].reshape(NW, 1, EPW))
    degp4 = degp.reshape(NW, _G, 1, _R)
    y = _p2_scale_matmul(x, degp4, W)
    zp = _p3_scatter(y, col3, row3)
    return _p4_combine(zp, degp4)
